# baseline (speedup 1.0000x reference)
import functools

import jax
import jax.numpy as jnp
from jax import lax
from jax.experimental import pallas as pl
from jax.experimental.pallas import tpu as pltpu
from jax.experimental.pallas import tpu_sc as plsc


def _gather_kernel(B, H, D, num_workers, cb):
    bat_w = B // num_workers
    n_chunks = bat_w // cb
    mesh = plsc.VectorSubcoreMesh(core_axis_name="c", subcore_axis_name="s")

    @functools.partial(
        pl.kernel,
        mesh=mesh,
        out_type=jax.ShapeDtypeStruct((B, H, D), jnp.float32),
        scratch_types=[
            pltpu.VMEM((bat_w, H), jnp.int32),
            pltpu.VMEM((cb, H, D), jnp.float32),
            pltpu.SemaphoreType.DMA,
        ],
        compiler_params=pltpu.CompilerParams(use_tc_tiling_on_sc=False),
    )
    def k(idx_hbm, table_hbm, out_hbm, idx_v, rows_v, sem):
        nc = lax.axis_size("c")
        wid = lax.axis_index("s") * nc + lax.axis_index("c")
        bbase = wid * bat_w
        pltpu.sync_copy(idx_hbm.at[pl.ds(bbase, bat_w)], idx_v)

        def body(c, carry):
            for j in range(cb):
                pltpu.async_copy(
                    table_hbm.at[idx_v.at[c * cb + j]], rows_v.at[j], sem
                )
            for j in range(cb):
                pltpu.make_async_copy(
                    table_hbm.at[idx_v.at[0]], rows_v.at[j], sem
                ).wait()
            pltpu.sync_copy(rows_v, out_hbm.at[pl.ds(bbase + c * cb, cb)])
            return carry

        lax.fori_loop(0, n_chunks, body, 0)

    return k


def kernel(indexs, weights):
    B, H = indexs.shape
    V, D = weights.shape
    idx = indexs.astype(jnp.int32)
    nsplit = 4
    part = B // nsplit
    gk = _gather_kernel(part, H, D, 32, 32)
    outs = [gk(idx[i * part : (i + 1) * part], weights) for i in range(nsplit)]
    return jnp.concatenate(outs, axis=0)


# speedup vs baseline: 1.0060x; 1.0060x over previous
import functools

import jax
import jax.numpy as jnp
from jax import lax
from jax.experimental import pallas as pl
from jax.experimental.pallas import tpu as pltpu
from jax.experimental.pallas import tpu_sc as plsc


def _gather_kernel(B, H, D, num_workers, cb):
    bat_w = B // num_workers
    n_chunks = bat_w // cb
    mesh = plsc.VectorSubcoreMesh(core_axis_name="c", subcore_axis_name="s")

    @functools.partial(
        pl.kernel,
        mesh=mesh,
        out_type=jax.ShapeDtypeStruct((B, H, D), jnp.float32),
        scratch_types=[
            pltpu.VMEM((bat_w, H), jnp.int32),
            pltpu.VMEM((cb, H, D), jnp.float32),
            pltpu.SemaphoreType.DMA,
        ],
        compiler_params=pltpu.CompilerParams(use_tc_tiling_on_sc=False),
    )
    def k(idx_hbm, table_hbm, out_hbm, idx_v, rows_v, sem):
        nc = lax.axis_size("c")
        wid = lax.axis_index("s") * nc + lax.axis_index("c")
        bbase = wid * bat_w
        pltpu.sync_copy(idx_hbm.at[pl.ds(bbase, bat_w)], idx_v)

        def body(c, carry):
            for j in range(cb):
                pltpu.async_copy(
                    table_hbm.at[idx_v.at[c * cb + j]], rows_v.at[j], sem
                )
            for j in range(cb):
                pltpu.make_async_copy(
                    table_hbm.at[idx_v.at[0]], rows_v.at[j], sem
                ).wait()
            pltpu.sync_copy(rows_v, out_hbm.at[pl.ds(bbase + c * cb, cb)])
            return carry

        lax.fori_loop(0, n_chunks, body, 0)

    return k


def kernel(indexs, weights):
    B, H = indexs.shape
    V, D = weights.shape
    idx = indexs.astype(jnp.int32)
    nsplit = 4
    part = B // nsplit
    gk = _gather_kernel(part, H, D, 32, 64)
    outs = [gk(idx[i * part : (i + 1) * part], weights) for i in range(nsplit)]
    return jnp.concatenate(outs, axis=0)
